# initial kernel scaffold (unmeasured)
import jax
import jax.numpy as jnp
from jax import lax
from jax.experimental import pallas as pl
from jax.experimental.pallas import tpu as pltpu

N_DEV = 16
SQ = 1024
SKV = 1024
DH = 128
H_LOC = 8
D_LOC = H_LOC * DH
CHUNK = SQ // N_DEV
SCALE = 0.08838834764831843
BLK = 64


def _body(x_ref, wq_ref, k_ref, v_ref, wo_ref, out_ref,
          ctx_ref, acc_ref, rs_buf,
          rs_send, rs_recv, ag_send, ag_recv):
    d = lax.axis_index("i")
    right = lax.rem(d + 1, N_DEV)

    q = jnp.dot(x_ref[...], wq_ref[...], preferred_element_type=jnp.float32)
    qb = q.astype(jnp.bfloat16).reshape(SQ, H_LOC, DH)

    ri = lax.broadcasted_iota(jnp.int32, (SQ, SKV), 0)
    ci = lax.broadcasted_iota(jnp.int32, (SQ, SKV), 1)
    mask = (ri // BLK) >= (ci // BLK)

    for h in range(H_LOC):
        q_h = qb[:, h, :]
        k_h = k_ref[:, h, :]
        v_h = v_ref[:, h, :]
        s = lax.dot_general(q_h, k_h, (((1,), (1,)), ((), ())),
                            preferred_element_type=jnp.float32) * SCALE
        s = jnp.where(mask, s, -1e9)
        m = jnp.max(s, axis=1, keepdims=True)
        w = jnp.exp(s - m)
        denom = jnp.sum(w, axis=1, keepdims=True)
        ctx_h = lax.dot_general(w.astype(jnp.bfloat16), v_h,
                                (((1,), (0,)), ((), ())),
                                preferred_element_type=jnp.float32)
        ctx_ref[:, h * DH:(h + 1) * DH] = (ctx_h / denom).astype(jnp.bfloat16)

    acc_ref[...] = jnp.dot(ctx_ref[...], wo_ref[...],
                           preferred_element_type=jnp.float32)

    for s_ in range(N_DEV - 1):
        c_send = lax.rem(d + N_DEV - s_, N_DEV)
        c_recv = lax.rem(d + N_DEV - s_ - 1, N_DEV)
        rdma = pltpu.make_async_remote_copy(
            src_ref=acc_ref.at[pl.ds(c_send * CHUNK, CHUNK), :],
            dst_ref=rs_buf.at[s_],
            send_sem=rs_send.at[s_],
            recv_sem=rs_recv.at[s_],
            device_id=(right,),
            device_id_type=pl.DeviceIdType.MESH,
        )
        rdma.start()
        rdma.wait()
        acc_ref[pl.ds(c_recv * CHUNK, CHUNK), :] = (
            acc_ref[pl.ds(c_recv * CHUNK, CHUNK), :] + rs_buf[s_]
        )

    own = lax.rem(d + 1, N_DEV)
    out_ref[pl.ds(own * CHUNK, CHUNK), :] = acc_ref[pl.ds(own * CHUNK, CHUNK), :]

    for s_ in range(N_DEV - 1):
        c_send = lax.rem(d + 1 + N_DEV - s_, N_DEV)
        rdma = pltpu.make_async_remote_copy(
            src_ref=out_ref.at[pl.ds(c_send * CHUNK, CHUNK), :],
            dst_ref=out_ref.at[pl.ds(c_send * CHUNK, CHUNK), :],
            send_sem=ag_send.at[s_],
            recv_sem=ag_recv.at[s_],
            device_id=(right,),
            device_id_type=pl.DeviceIdType.MESH,
        )
        rdma.start()
        rdma.wait()


def kernel(x, Wq, K_ext, V_ext, Wo):
    i = lax.axis_index("i")
    xb = x[0].astype(jnp.bfloat16)
    wqb = Wq.astype(jnp.bfloat16)
    wob = Wo.astype(jnp.bfloat16)
    k_sl = lax.dynamic_slice_in_dim(K_ext[0], i * H_LOC, H_LOC, axis=1)
    v_sl = lax.dynamic_slice_in_dim(V_ext[0], i * H_LOC, H_LOC, axis=1)
    k_sl = k_sl.astype(jnp.bfloat16)
    v_sl = v_sl.astype(jnp.bfloat16)

    out = pl.pallas_call(
        _body,
        out_shape=jax.ShapeDtypeStruct((SQ, SQ), jnp.float32),
        in_specs=[pl.BlockSpec(memory_space=pltpu.VMEM)] * 5,
        out_specs=pl.BlockSpec(memory_space=pltpu.VMEM),
        scratch_shapes=[
            pltpu.VMEM((SQ, D_LOC), jnp.bfloat16),
            pltpu.VMEM((SQ, SQ), jnp.float32),
            pltpu.VMEM((N_DEV - 1, CHUNK, SQ), jnp.float32),
            pltpu.SemaphoreType.DMA((N_DEV - 1,)),
            pltpu.SemaphoreType.DMA((N_DEV - 1,)),
            pltpu.SemaphoreType.DMA((N_DEV - 1,)),
            pltpu.SemaphoreType.DMA((N_DEV - 1,)),
        ],
        compiler_params=pltpu.CompilerParams(collective_id=0),
    )(xb, wqb, k_sl, v_sl, wob)
    return out[None, :, :]


# baseline (device time: 200759 ns/iter reference)
import jax
import jax.numpy as jnp
from jax import lax
from jax.experimental import pallas as pl
from jax.experimental.pallas import tpu as pltpu

N_DEV = 16
SQ = 1024
SKV = 1024
DH = 128
H_LOC = 8
D_LOC = H_LOC * DH
CHUNK = SQ // N_DEV
SCALE = 0.08838834764831843
BLK = 64


def _body(x_ref, wq_ref, k_ref, v_ref, wo_ref, out_ref,
          ctx_ref, acc_ref, rs_buf,
          rs_send, rs_recv, ag_send, ag_recv):
    d = lax.axis_index("i")
    right = lax.rem(d + 1, N_DEV)

    q = jnp.dot(x_ref[...], wq_ref[...], preferred_element_type=jnp.float32)
    qb = q.astype(jnp.bfloat16).reshape(SQ, H_LOC, DH)

    ri = lax.broadcasted_iota(jnp.int32, (SQ, SKV), 0)
    ci = lax.broadcasted_iota(jnp.int32, (SQ, SKV), 1)
    mask = (ri // BLK) >= (ci // BLK)

    for h in range(H_LOC):
        q_h = qb[:, h, :]
        k_h = k_ref[:, h, :]
        v_h = v_ref[:, h, :]
        s = lax.dot_general(q_h, k_h, (((1,), (1,)), ((), ())),
                            preferred_element_type=jnp.float32) * SCALE
        s = jnp.where(mask, s, -1e9)
        m = jnp.max(s, axis=1, keepdims=True)
        w = jnp.exp(s - m)
        denom = jnp.sum(w, axis=1, keepdims=True)
        ctx_h = lax.dot_general(w.astype(jnp.bfloat16), v_h,
                                (((1,), (0,)), ((), ())),
                                preferred_element_type=jnp.float32)
        ctx_ref[:, h * DH:(h + 1) * DH] = (ctx_h / denom).astype(jnp.bfloat16)

    acc_ref[...] = jnp.dot(ctx_ref[...], wo_ref[...],
                           preferred_element_type=jnp.float32)

    for s_ in range(N_DEV - 1):
        c_send = lax.rem(d + N_DEV - s_, N_DEV)
        c_recv = lax.rem(d + N_DEV - s_ - 1, N_DEV)
        rdma = pltpu.make_async_remote_copy(
            src_ref=acc_ref.at[pl.ds(c_send * CHUNK, CHUNK), :],
            dst_ref=rs_buf.at[s_],
            send_sem=rs_send.at[s_],
            recv_sem=rs_recv.at[s_],
            device_id=(right,),
            device_id_type=pl.DeviceIdType.MESH,
        )
        rdma.start()
        rdma.wait()
        acc_ref[pl.ds(c_recv * CHUNK, CHUNK), :] = (
            acc_ref[pl.ds(c_recv * CHUNK, CHUNK), :] + rs_buf[s_]
        )

    own = lax.rem(d + 1, N_DEV)
    out_ref[pl.ds(own * CHUNK, CHUNK), :] = acc_ref[pl.ds(own * CHUNK, CHUNK), :]

    for s_ in range(N_DEV - 1):
        c_send = lax.rem(d + 1 + N_DEV - s_, N_DEV)
        rdma = pltpu.make_async_remote_copy(
            src_ref=out_ref.at[pl.ds(c_send * CHUNK, CHUNK), :],
            dst_ref=out_ref.at[pl.ds(c_send * CHUNK, CHUNK), :],
            send_sem=ag_send.at[s_],
            recv_sem=ag_recv.at[s_],
            device_id=(right,),
            device_id_type=pl.DeviceIdType.MESH,
        )
        rdma.start()
        rdma.wait()


def kernel(x, Wq, K_ext, V_ext, Wo):
    i = lax.axis_index("i")
    xb = x[0].astype(jnp.bfloat16)
    wqb = Wq.astype(jnp.bfloat16)
    wob = Wo.astype(jnp.bfloat16)
    k_sl = lax.dynamic_slice_in_dim(K_ext[0], i * H_LOC, H_LOC, axis=1)
    v_sl = lax.dynamic_slice_in_dim(V_ext[0], i * H_LOC, H_LOC, axis=1)
    k_sl = k_sl.astype(jnp.bfloat16)
    v_sl = v_sl.astype(jnp.bfloat16)

    out = pl.pallas_call(
        _body,
        out_shape=jax.ShapeDtypeStruct((SQ, SQ), jnp.float32),
        in_specs=[pl.BlockSpec(memory_space=pltpu.VMEM)] * 5,
        out_specs=pl.BlockSpec(memory_space=pltpu.VMEM),
        scratch_shapes=[
            pltpu.VMEM((SQ, D_LOC), jnp.bfloat16),
            pltpu.VMEM((SQ, SQ), jnp.float32),
            pltpu.VMEM((N_DEV - 1, CHUNK, SQ), jnp.float32),
            pltpu.SemaphoreType.DMA((N_DEV - 1,)),
            pltpu.SemaphoreType.DMA((N_DEV - 1,)),
            pltpu.SemaphoreType.DMA((N_DEV - 1,)),
            pltpu.SemaphoreType.DMA((N_DEV - 1,)),
        ],
        compiler_params=pltpu.CompilerParams(
            vmem_limit_bytes=100 * 1024 * 1024,
        ),
    )(xb, wqb, k_sl, v_sl, wob)
    return out[None, :, :]


# device time: 103547 ns/iter; 1.9388x vs baseline; 1.9388x over previous
import jax
import jax.numpy as jnp
from jax import lax
from jax.experimental import pallas as pl
from jax.experimental.pallas import tpu as pltpu

N_DEV = 16
SQ = 1024
SKV = 1024
DH = 128
H_LOC = 8
D_LOC = H_LOC * DH
CHUNK = SQ // N_DEV
SCALE = 0.08838834764831843
BLK = 64


def _body(x_ref, wq_ref, k_ref, v_ref, wo_ref, out_ref,
          ctx_ref, acc_ref, stage_ref, red_ref, p1_buf,
          p1_send, p1_recv, p2_send, p2_recv):
    d = lax.axis_index("i")

    q = jnp.dot(x_ref[...], wq_ref[...], preferred_element_type=jnp.float32)
    qb = q.astype(jnp.bfloat16).reshape(SQ, H_LOC, DH)

    ri = lax.broadcasted_iota(jnp.int32, (SQ, SKV), 0)
    ci = lax.broadcasted_iota(jnp.int32, (SQ, SKV), 1)
    mask = (ri // BLK) >= (ci // BLK)

    for h in range(H_LOC):
        q_h = qb[:, h, :]
        k_h = k_ref[:, h, :]
        v_h = v_ref[:, h, :]
        s = lax.dot_general(q_h, k_h, (((1,), (1,)), ((), ())),
                            preferred_element_type=jnp.float32) * SCALE
        s = jnp.where(mask, s, -1e9)
        m = jnp.max(s, axis=1, keepdims=True)
        w = jnp.exp(s - m)
        denom = jnp.sum(w, axis=1, keepdims=True)
        ctx_h = lax.dot_general(w.astype(jnp.bfloat16), v_h,
                                (((1,), (0,)), ((), ())),
                                preferred_element_type=jnp.float32)
        ctx_ref[:, h * DH:(h + 1) * DH] = (ctx_h / denom).astype(jnp.bfloat16)

    acc_ref[...] = jnp.dot(ctx_ref[...], wo_ref[...],
                           preferred_element_type=jnp.float32)
    stage_ref[...] = acc_ref[...].astype(jnp.bfloat16)

    p1 = []
    for off in range(1, N_DEV):
        p = lax.rem(d + off, N_DEV)
        rdma = pltpu.make_async_remote_copy(
            src_ref=stage_ref.at[pl.ds(p * CHUNK, CHUNK), :],
            dst_ref=p1_buf.at[off - 1],
            send_sem=p1_send.at[off - 1],
            recv_sem=p1_recv.at[off - 1],
            device_id=(p,),
            device_id_type=pl.DeviceIdType.MESH,
        )
        rdma.start()
        p1.append(rdma)

    red = acc_ref[pl.ds(d * CHUNK, CHUNK), :]
    for off in range(1, N_DEV):
        p1[off - 1].wait_recv()
        red = red + p1_buf[off - 1].astype(jnp.float32)
    red_ref[...] = red.astype(jnp.bfloat16)
    out_ref[pl.ds(d * CHUNK, CHUNK), :] = red_ref[...]

    p2 = []
    for off in range(1, N_DEV):
        p = lax.rem(d + off, N_DEV)
        rdma = pltpu.make_async_remote_copy(
            src_ref=red_ref,
            dst_ref=out_ref.at[pl.ds(d * CHUNK, CHUNK), :],
            send_sem=p2_send.at[off - 1],
            recv_sem=p2_recv.at[off - 1],
            device_id=(p,),
            device_id_type=pl.DeviceIdType.MESH,
        )
        rdma.start()
        p2.append(rdma)

    for off in range(1, N_DEV):
        src = lax.rem(d + N_DEV - off, N_DEV)
        rdma = pltpu.make_async_remote_copy(
            src_ref=red_ref,
            dst_ref=out_ref.at[pl.ds(src * CHUNK, CHUNK), :],
            send_sem=p1_send.at[off - 1],
            recv_sem=p2_recv.at[off - 1],
            device_id=(src,),
            device_id_type=pl.DeviceIdType.MESH,
        )
        rdma.wait_recv()

    for off in range(1, N_DEV):
        p1[off - 1].wait_send()
        p2[off - 1].wait_send()


def kernel(x, Wq, K_ext, V_ext, Wo):
    i = lax.axis_index("i")
    xb = x[0].astype(jnp.bfloat16)
    wqb = Wq.astype(jnp.bfloat16)
    wob = Wo.astype(jnp.bfloat16)
    k_sl = lax.dynamic_slice_in_dim(K_ext[0], i * H_LOC, H_LOC, axis=1)
    v_sl = lax.dynamic_slice_in_dim(V_ext[0], i * H_LOC, H_LOC, axis=1)
    k_sl = k_sl.astype(jnp.bfloat16)
    v_sl = v_sl.astype(jnp.bfloat16)

    out = pl.pallas_call(
        _body,
        out_shape=jax.ShapeDtypeStruct((SQ, SQ), jnp.bfloat16),
        in_specs=[pl.BlockSpec(memory_space=pltpu.VMEM)] * 5,
        out_specs=pl.BlockSpec(memory_space=pltpu.VMEM),
        scratch_shapes=[
            pltpu.VMEM((SQ, D_LOC), jnp.bfloat16),
            pltpu.VMEM((SQ, SQ), jnp.float32),
            pltpu.VMEM((SQ, SQ), jnp.bfloat16),
            pltpu.VMEM((CHUNK, SQ), jnp.bfloat16),
            pltpu.VMEM((N_DEV - 1, CHUNK, SQ), jnp.bfloat16),
            pltpu.SemaphoreType.DMA((N_DEV - 1,)),
            pltpu.SemaphoreType.DMA((N_DEV - 1,)),
            pltpu.SemaphoreType.DMA((N_DEV - 1,)),
            pltpu.SemaphoreType.DMA((N_DEV - 1,)),
        ],
        compiler_params=pltpu.CompilerParams(
            vmem_limit_bytes=100 * 1024 * 1024,
        ),
    )(xb, wqb, k_sl, v_sl, wob)
    return out[None, :, :]


# device time: 95920 ns/iter; 2.0930x vs baseline; 1.0795x over previous
import jax
import jax.numpy as jnp
from jax import lax
from jax.experimental import pallas as pl
from jax.experimental.pallas import tpu as pltpu

N_DEV = 16
SQ = 1024
SKV = 1024
DH = 128
H_LOC = 8
D_LOC = H_LOC * DH
CHUNK = SQ // N_DEV
HALF = SQ // 2
SCALE = 0.08838834764831843
BLK = 64


def _attn_band(q_band, k_all, v_all, mask_half, banded):
    if banded:
        k, v = k_all[:, :HALF, :], v_all[:, :HALF, :]
        s = lax.dot_general(q_band, k, (((2,), (2,)), ((0,), (0,))),
                            preferred_element_type=jnp.float32) * SCALE
        w = jnp.where(mask_half[None], jnp.exp(s), 0.0)
        denom = jnp.sum(w, axis=2, keepdims=True)
        ctx = lax.dot_general(w.astype(jnp.bfloat16), v,
                              (((2,), (1,)), ((0,), (0,))),
                              preferred_element_type=jnp.float32)
        return ctx / denom
    s = lax.dot_general(q_band, k_all, (((2,), (2,)), ((0,), (0,))),
                        preferred_element_type=jnp.float32) * SCALE
    w_l = jnp.exp(s[:, :, :HALF])
    w_r = jnp.where(mask_half[None], jnp.exp(s[:, :, HALF:]), 0.0)
    denom = (jnp.sum(w_l, axis=2, keepdims=True)
             + jnp.sum(w_r, axis=2, keepdims=True))
    ctx = (lax.dot_general(w_l.astype(jnp.bfloat16), v_all[:, :HALF, :],
                           (((2,), (1,)), ((0,), (0,))),
                           preferred_element_type=jnp.float32)
           + lax.dot_general(w_r.astype(jnp.bfloat16), v_all[:, HALF:, :],
                             (((2,), (1,)), ((0,), (0,))),
                             preferred_element_type=jnp.float32))
    return ctx / denom


def _body(x_ref, wq_ref, k_ref, v_ref, wo_ref, out_ref,
          ctx_ref, red_ref, stage_ref, p1_buf,
          p1_send, p1_recv, p2_send, p2_recv):
    d = lax.axis_index("i")

    q = jnp.dot(x_ref[...], wq_ref[...], preferred_element_type=jnp.float32)
    q_all = q.astype(jnp.bfloat16).reshape(SQ, H_LOC, DH).transpose(1, 0, 2)

    ri = lax.broadcasted_iota(jnp.int32, (HALF, HALF), 0)
    ci = lax.broadcasted_iota(jnp.int32, (HALF, HALF), 1)
    mask_half = (ri // BLK) >= (ci // BLK)

    k_all = k_ref[...]
    v_all = v_ref[...]

    ctx_a = _attn_band(q_all[:, :HALF, :], k_all, v_all, mask_half, True)
    ctx_ref[:HALF, :] = (
        ctx_a.astype(jnp.bfloat16).transpose(1, 0, 2).reshape(HALF, D_LOC))
    ctx_b = _attn_band(q_all[:, HALF:, :], k_all, v_all, mask_half, False)
    ctx_ref[HALF:, :] = (
        ctx_b.astype(jnp.bfloat16).transpose(1, 0, 2).reshape(HALF, D_LOC))

    p1 = []
    for off in range(1, N_DEV):
        p = lax.rem(d + off, N_DEV)
        part = jnp.dot(ctx_ref[pl.ds(p * CHUNK, CHUNK), :], wo_ref[...],
                       preferred_element_type=jnp.float32)
        stage_ref[off - 1] = part.astype(jnp.bfloat16)
        rdma = pltpu.make_async_remote_copy(
            src_ref=stage_ref.at[off - 1],
            dst_ref=p1_buf.at[off - 1],
            send_sem=p1_send.at[off - 1],
            recv_sem=p1_recv.at[off - 1],
            device_id=(p,),
            device_id_type=pl.DeviceIdType.MESH,
        )
        rdma.start()
        p1.append(rdma)

    red = jnp.dot(ctx_ref[pl.ds(d * CHUNK, CHUNK), :], wo_ref[...],
                  preferred_element_type=jnp.float32)
    for off in range(1, N_DEV):
        p1[off - 1].wait_recv()
        red = red + p1_buf[off - 1].astype(jnp.float32)
    red_ref[...] = red.astype(jnp.bfloat16)
    out_ref[pl.ds(d * CHUNK, CHUNK), :] = red_ref[...]

    p2 = []
    for off in range(1, N_DEV):
        p = lax.rem(d + off, N_DEV)
        rdma = pltpu.make_async_remote_copy(
            src_ref=red_ref,
            dst_ref=out_ref.at[pl.ds(d * CHUNK, CHUNK), :],
            send_sem=p2_send.at[off - 1],
            recv_sem=p2_recv.at[off - 1],
            device_id=(p,),
            device_id_type=pl.DeviceIdType.MESH,
        )
        rdma.start()
        p2.append(rdma)

    for off in range(1, N_DEV):
        src = lax.rem(d + N_DEV - off, N_DEV)
        rdma = pltpu.make_async_remote_copy(
            src_ref=red_ref,
            dst_ref=out_ref.at[pl.ds(src * CHUNK, CHUNK), :],
            send_sem=p1_send.at[off - 1],
            recv_sem=p2_recv.at[off - 1],
            device_id=(src,),
            device_id_type=pl.DeviceIdType.MESH,
        )
        rdma.wait_recv()

    for off in range(1, N_DEV):
        p1[off - 1].wait_send()
        p2[off - 1].wait_send()


def kernel(x, Wq, K_ext, V_ext, Wo):
    i = lax.axis_index("i")
    xb = x[0].astype(jnp.bfloat16)
    wqb = Wq.astype(jnp.bfloat16)
    wob = Wo.astype(jnp.bfloat16)
    k_sl = lax.dynamic_slice_in_dim(K_ext[0], i * H_LOC, H_LOC, axis=1)
    v_sl = lax.dynamic_slice_in_dim(V_ext[0], i * H_LOC, H_LOC, axis=1)
    k_sl = k_sl.astype(jnp.bfloat16).transpose(1, 0, 2)
    v_sl = v_sl.astype(jnp.bfloat16).transpose(1, 0, 2)

    out = pl.pallas_call(
        _body,
        out_shape=jax.ShapeDtypeStruct((SQ, SQ), jnp.bfloat16),
        in_specs=[pl.BlockSpec(memory_space=pltpu.VMEM)] * 5,
        out_specs=pl.BlockSpec(memory_space=pltpu.VMEM),
        scratch_shapes=[
            pltpu.VMEM((SQ, D_LOC), jnp.bfloat16),
            pltpu.VMEM((CHUNK, SQ), jnp.bfloat16),
            pltpu.VMEM((N_DEV - 1, CHUNK, SQ), jnp.bfloat16),
            pltpu.VMEM((N_DEV - 1, CHUNK, SQ), jnp.bfloat16),
            pltpu.SemaphoreType.DMA((N_DEV - 1,)),
            pltpu.SemaphoreType.DMA((N_DEV - 1,)),
            pltpu.SemaphoreType.DMA((N_DEV - 1,)),
            pltpu.SemaphoreType.DMA((N_DEV - 1,)),
        ],
        compiler_params=pltpu.CompilerParams(
            vmem_limit_bytes=100 * 1024 * 1024,
        ),
    )(xb, wqb, k_sl, v_sl, wob)
    return out[None, :, :]


# device time: 82672 ns/iter; 2.4284x vs baseline; 1.1602x over previous
import jax
import jax.numpy as jnp
from jax import lax
from jax.experimental import pallas as pl
from jax.experimental.pallas import tpu as pltpu

N_DEV = 16
SQ = 1024
SKV = 1024
DH = 128
H_LOC = 8
D_LOC = H_LOC * DH
CHUNK = SQ // N_DEV
HALF = SQ // 2
SCALE = 0.08838834764831843
BLK = 64


def _attn_band(q_band, k_all, v_all, mask_half, banded):
    if banded:
        k, v = k_all[:, :HALF, :], v_all[:, :HALF, :]
        s = lax.dot_general(q_band, k, (((2,), (2,)), ((0,), (0,))),
                            preferred_element_type=jnp.float32) * SCALE
        w = jnp.where(mask_half[None], jnp.exp(s), 0.0)
        denom = jnp.sum(w, axis=2, keepdims=True)
        ctx = lax.dot_general(w.astype(jnp.bfloat16), v,
                              (((2,), (1,)), ((0,), (0,))),
                              preferred_element_type=jnp.float32)
        return ctx / denom
    s = lax.dot_general(q_band, k_all, (((2,), (2,)), ((0,), (0,))),
                        preferred_element_type=jnp.float32) * SCALE
    w_l = jnp.exp(s[:, :, :HALF])
    w_r = jnp.where(mask_half[None], jnp.exp(s[:, :, HALF:]), 0.0)
    denom = (jnp.sum(w_l, axis=2, keepdims=True)
             + jnp.sum(w_r, axis=2, keepdims=True))
    ctx = (lax.dot_general(w_l.astype(jnp.bfloat16), v_all[:, :HALF, :],
                           (((2,), (1,)), ((0,), (0,))),
                           preferred_element_type=jnp.float32)
           + lax.dot_general(w_r.astype(jnp.bfloat16), v_all[:, HALF:, :],
                             (((2,), (1,)), ((0,), (0,))),
                             preferred_element_type=jnp.float32))
    return ctx / denom


def _body(x_ref, wq_ref, k_hbm, v_hbm, wo_ref, out_ref,
          kf_ref, vf_ref, kb_ref, vb_ref, wob_ref, ctx_ref, red_ref,
          stage_ref, p1_buf,
          k_sems, v_sems, p1_send, p1_recv, p2_send, p2_recv):
    d = lax.axis_index("i")

    kv_copies = []
    for h in range(H_LOC):
        idx = d * H_LOC + h
        ck = pltpu.make_async_copy(k_hbm.at[0, :, idx, :], kf_ref.at[h],
                                   k_sems.at[h])
        cv = pltpu.make_async_copy(v_hbm.at[0, :, idx, :], vf_ref.at[h],
                                   v_sems.at[h])
        ck.start()
        cv.start()
        kv_copies += [ck, cv]

    q = jnp.dot(x_ref[0].astype(jnp.bfloat16), wq_ref[...].astype(jnp.bfloat16),
                preferred_element_type=jnp.float32)
    q_all = q.astype(jnp.bfloat16).reshape(SQ, H_LOC, DH).transpose(1, 0, 2)
    wob_ref[...] = wo_ref[...].astype(jnp.bfloat16)

    ri = lax.broadcasted_iota(jnp.int32, (HALF, HALF), 0)
    ci = lax.broadcasted_iota(jnp.int32, (HALF, HALF), 1)
    mask_half = (ri // BLK) >= (ci // BLK)

    for cp in kv_copies:
        cp.wait()
    kb_ref[...] = kf_ref[...].astype(jnp.bfloat16)
    vb_ref[...] = vf_ref[...].astype(jnp.bfloat16)
    k_all = kb_ref[...]
    v_all = vb_ref[...]

    ctx_a = _attn_band(q_all[:, :HALF, :], k_all, v_all, mask_half, True)
    ctx_ref[:HALF, :] = (
        ctx_a.astype(jnp.bfloat16).transpose(1, 0, 2).reshape(HALF, D_LOC))
    ctx_b = _attn_band(q_all[:, HALF:, :], k_all, v_all, mask_half, False)
    ctx_ref[HALF:, :] = (
        ctx_b.astype(jnp.bfloat16).transpose(1, 0, 2).reshape(HALF, D_LOC))

    p1 = []
    for off in range(1, N_DEV):
        p = lax.rem(d + off, N_DEV)
        part = jnp.dot(ctx_ref[pl.ds(p * CHUNK, CHUNK), :], wob_ref[...],
                       preferred_element_type=jnp.float32)
        stage_ref[off - 1] = part.astype(jnp.bfloat16)
        rdma = pltpu.make_async_remote_copy(
            src_ref=stage_ref.at[off - 1],
            dst_ref=p1_buf.at[off - 1],
            send_sem=p1_send.at[off - 1],
            recv_sem=p1_recv.at[off - 1],
            device_id=(p,),
            device_id_type=pl.DeviceIdType.MESH,
        )
        rdma.start()
        p1.append(rdma)

    red = jnp.dot(ctx_ref[pl.ds(d * CHUNK, CHUNK), :], wob_ref[...],
                  preferred_element_type=jnp.float32)
    for off in range(1, N_DEV):
        p1[off - 1].wait_recv()
        red = red + p1_buf[off - 1].astype(jnp.float32)
    red_ref[...] = red.astype(jnp.bfloat16)
    out_ref[pl.ds(d * CHUNK, CHUNK), :] = red_ref[...]

    p2 = []
    for off in range(1, N_DEV):
        p = lax.rem(d + off, N_DEV)
        rdma = pltpu.make_async_remote_copy(
            src_ref=red_ref,
            dst_ref=out_ref.at[pl.ds(d * CHUNK, CHUNK), :],
            send_sem=p2_send.at[off - 1],
            recv_sem=p2_recv.at[off - 1],
            device_id=(p,),
            device_id_type=pl.DeviceIdType.MESH,
        )
        rdma.start()
        p2.append(rdma)

    for off in range(1, N_DEV):
        src = lax.rem(d + N_DEV - off, N_DEV)
        rdma = pltpu.make_async_remote_copy(
            src_ref=red_ref,
            dst_ref=out_ref.at[pl.ds(src * CHUNK, CHUNK), :],
            send_sem=p1_send.at[off - 1],
            recv_sem=p2_recv.at[off - 1],
            device_id=(src,),
            device_id_type=pl.DeviceIdType.MESH,
        )
        rdma.wait_recv()

    for off in range(1, N_DEV):
        p1[off - 1].wait_send()
        p2[off - 1].wait_send()


def kernel(x, Wq, K_ext, V_ext, Wo):
    out = pl.pallas_call(
        _body,
        out_shape=jax.ShapeDtypeStruct((SQ, SQ), jnp.bfloat16),
        in_specs=[
            pl.BlockSpec(memory_space=pltpu.VMEM),
            pl.BlockSpec(memory_space=pltpu.VMEM),
            pl.BlockSpec(memory_space=pl.ANY),
            pl.BlockSpec(memory_space=pl.ANY),
            pl.BlockSpec(memory_space=pltpu.VMEM),
        ],
        out_specs=pl.BlockSpec(memory_space=pltpu.VMEM),
        scratch_shapes=[
            pltpu.VMEM((H_LOC, SKV, DH), jnp.float32),
            pltpu.VMEM((H_LOC, SKV, DH), jnp.float32),
            pltpu.VMEM((H_LOC, SKV, DH), jnp.bfloat16),
            pltpu.VMEM((H_LOC, SKV, DH), jnp.bfloat16),
            pltpu.VMEM((SQ, SQ), jnp.bfloat16),
            pltpu.VMEM((SQ, D_LOC), jnp.bfloat16),
            pltpu.VMEM((CHUNK, SQ), jnp.bfloat16),
            pltpu.VMEM((N_DEV - 1, CHUNK, SQ), jnp.bfloat16),
            pltpu.VMEM((N_DEV - 1, CHUNK, SQ), jnp.bfloat16),
            pltpu.SemaphoreType.DMA((H_LOC,)),
            pltpu.SemaphoreType.DMA((H_LOC,)),
            pltpu.SemaphoreType.DMA((N_DEV - 1,)),
            pltpu.SemaphoreType.DMA((N_DEV - 1,)),
            pltpu.SemaphoreType.DMA((N_DEV - 1,)),
            pltpu.SemaphoreType.DMA((N_DEV - 1,)),
        ],
        compiler_params=pltpu.CompilerParams(
            vmem_limit_bytes=110 * 1024 * 1024,
        ),
    )(x, Wq, K_ext, V_ext, Wo)
    return out[None, :, :]


# device time: 76976 ns/iter; 2.6081x vs baseline; 1.0740x over previous
import jax
import jax.numpy as jnp
from jax import lax
from jax.experimental import pallas as pl
from jax.experimental.pallas import tpu as pltpu

N_DEV = 16
SQ = 1024
SKV = 1024
DH = 128
H_LOC = 8
D_LOC = H_LOC * DH
STRIP = 32
HALF = SQ // 2
SCALE = 0.08838834764831843
BLK = 64


def _attn_band(q_band, k_all, v_all, mask_half, banded):
    if banded:
        k, v = k_all[:, :HALF, :], v_all[:, :HALF, :]
        s = lax.dot_general(q_band, k, (((2,), (2,)), ((0,), (0,))),
                            preferred_element_type=jnp.float32) * SCALE
        w = jnp.where(mask_half[None], jnp.exp(s), 0.0)
        denom = jnp.sum(w, axis=2, keepdims=True)
        ctx = lax.dot_general(w.astype(jnp.bfloat16), v,
                              (((2,), (1,)), ((0,), (0,))),
                              preferred_element_type=jnp.float32)
        return ctx / denom
    s = lax.dot_general(q_band, k_all, (((2,), (2,)), ((0,), (0,))),
                        preferred_element_type=jnp.float32) * SCALE
    w_l = jnp.exp(s[:, :, :HALF])
    w_r = jnp.where(mask_half[None], jnp.exp(s[:, :, HALF:]), 0.0)
    denom = (jnp.sum(w_l, axis=2, keepdims=True)
             + jnp.sum(w_r, axis=2, keepdims=True))
    ctx = (lax.dot_general(w_l.astype(jnp.bfloat16), v_all[:, :HALF, :],
                           (((2,), (1,)), ((0,), (0,))),
                           preferred_element_type=jnp.float32)
           + lax.dot_general(w_r.astype(jnp.bfloat16), v_all[:, HALF:, :],
                             (((2,), (1,)), ((0,), (0,))),
                             preferred_element_type=jnp.float32))
    return ctx / denom


def _body(x_ref, wq_ref, k_hbm, v_hbm, wo_ref, out_ref,
          kf_ref, vf_ref, kb_ref, vb_ref, wob_ref, ctx_ref, red_ref,
          stage_ref, p1_buf,
          k_sems, v_sems, p1_send, p1_recv, p2_send, p2_recv):
    d = lax.axis_index("i")

    def strip_rows(band, dev):
        return pl.ds(band * HALF + dev * STRIP, STRIP)

    kv_copies = []
    for h in range(H_LOC):
        idx = d * H_LOC + h
        ck = pltpu.make_async_copy(k_hbm.at[0, :, idx, :], kf_ref.at[h],
                                   k_sems.at[h])
        cv = pltpu.make_async_copy(v_hbm.at[0, :, idx, :], vf_ref.at[h],
                                   v_sems.at[h])
        ck.start()
        cv.start()
        kv_copies += [ck, cv]

    q = jnp.dot(x_ref[0].astype(jnp.bfloat16), wq_ref[...].astype(jnp.bfloat16),
                preferred_element_type=jnp.float32)
    q_all = q.astype(jnp.bfloat16).reshape(SQ, H_LOC, DH).transpose(1, 0, 2)
    wob_ref[...] = wo_ref[...].astype(jnp.bfloat16)

    ri = lax.broadcasted_iota(jnp.int32, (HALF, HALF), 0)
    ci = lax.broadcasted_iota(jnp.int32, (HALF, HALF), 1)
    mask_half = (ri // BLK) >= (ci // BLK)

    for cp in kv_copies:
        cp.wait()
    kb_ref[...] = kf_ref[...].astype(jnp.bfloat16)
    vb_ref[...] = vf_ref[...].astype(jnp.bfloat16)
    k_all = kb_ref[...]
    v_all = vb_ref[...]

    def project(rows):
        return jnp.dot(ctx_ref[rows, :], wob_ref[...],
                       preferred_element_type=jnp.float32)

    def p1_sends(band):
        rdmas = []
        for off in range(1, N_DEV):
            p = lax.rem(d + off, N_DEV)
            stage_ref[band, off - 1] = (
                project(strip_rows(band, p)).astype(jnp.bfloat16))
            rdma = pltpu.make_async_remote_copy(
                src_ref=stage_ref.at[band, off - 1],
                dst_ref=p1_buf.at[band, off - 1],
                send_sem=p1_send.at[band, off - 1],
                recv_sem=p1_recv.at[band, off - 1],
                device_id=(p,),
                device_id_type=pl.DeviceIdType.MESH,
            )
            rdma.start()
            rdmas.append(rdma)
        return rdmas

    def reduce_and_broadcast(band, p1_rdmas):
        red = project(strip_rows(band, d))
        for off in range(1, N_DEV):
            p1_rdmas[off - 1].wait_recv()
            red = red + p1_buf[band, off - 1].astype(jnp.float32)
        red_ref[band] = red.astype(jnp.bfloat16)
        out_ref[strip_rows(band, d), :] = red_ref[band]
        rdmas = []
        for off in range(1, N_DEV):
            p = lax.rem(d + off, N_DEV)
            rdma = pltpu.make_async_remote_copy(
                src_ref=red_ref.at[band],
                dst_ref=out_ref.at[strip_rows(band, d), :],
                send_sem=p2_send.at[band, off - 1],
                recv_sem=p2_recv.at[band, off - 1],
                device_id=(p,),
                device_id_type=pl.DeviceIdType.MESH,
            )
            rdma.start()
            rdmas.append(rdma)
        return rdmas

    ctx_a = _attn_band(q_all[:, :HALF, :], k_all, v_all, mask_half, True)
    ctx_ref[:HALF, :] = (
        ctx_a.astype(jnp.bfloat16).transpose(1, 0, 2).reshape(HALF, D_LOC))
    p1_a = p1_sends(0)

    ctx_b = _attn_band(q_all[:, HALF:, :], k_all, v_all, mask_half, False)
    ctx_ref[HALF:, :] = (
        ctx_b.astype(jnp.bfloat16).transpose(1, 0, 2).reshape(HALF, D_LOC))
    p1_b = p1_sends(1)

    p2_a = reduce_and_broadcast(0, p1_a)
    p2_b = reduce_and_broadcast(1, p1_b)

    for band in range(2):
        for off in range(1, N_DEV):
            src = lax.rem(d + N_DEV - off, N_DEV)
            rdma = pltpu.make_async_remote_copy(
                src_ref=red_ref.at[band],
                dst_ref=out_ref.at[strip_rows(band, src), :],
                send_sem=p1_send.at[band, off - 1],
                recv_sem=p2_recv.at[band, off - 1],
                device_id=(src,),
                device_id_type=pl.DeviceIdType.MESH,
            )
            rdma.wait_recv()

    for rdma in p1_a + p1_b + p2_a + p2_b:
        rdma.wait_send()


def kernel(x, Wq, K_ext, V_ext, Wo):
    out = pl.pallas_call(
        _body,
        out_shape=jax.ShapeDtypeStruct((SQ, SQ), jnp.bfloat16),
        in_specs=[
            pl.BlockSpec(memory_space=pltpu.VMEM),
            pl.BlockSpec(memory_space=pltpu.VMEM),
            pl.BlockSpec(memory_space=pl.ANY),
            pl.BlockSpec(memory_space=pl.ANY),
            pl.BlockSpec(memory_space=pltpu.VMEM),
        ],
        out_specs=pl.BlockSpec(memory_space=pltpu.VMEM),
        scratch_shapes=[
            pltpu.VMEM((H_LOC, SKV, DH), jnp.float32),
            pltpu.VMEM((H_LOC, SKV, DH), jnp.float32),
            pltpu.VMEM((H_LOC, SKV, DH), jnp.bfloat16),
            pltpu.VMEM((H_LOC, SKV, DH), jnp.bfloat16),
            pltpu.VMEM((SQ, SQ), jnp.bfloat16),
            pltpu.VMEM((SQ, D_LOC), jnp.bfloat16),
            pltpu.VMEM((2, STRIP, SQ), jnp.bfloat16),
            pltpu.VMEM((2, N_DEV - 1, STRIP, SQ), jnp.bfloat16),
            pltpu.VMEM((2, N_DEV - 1, STRIP, SQ), jnp.bfloat16),
            pltpu.SemaphoreType.DMA((H_LOC,)),
            pltpu.SemaphoreType.DMA((H_LOC,)),
            pltpu.SemaphoreType.DMA((2, N_DEV - 1)),
            pltpu.SemaphoreType.DMA((2, N_DEV - 1)),
            pltpu.SemaphoreType.DMA((2, N_DEV - 1)),
            pltpu.SemaphoreType.DMA((2, N_DEV - 1)),
        ],
        compiler_params=pltpu.CompilerParams(
            vmem_limit_bytes=110 * 1024 * 1024,
        ),
    )(x, Wq, K_ext, V_ext, Wo)
    return out[None, :, :]
